# baseline (speedup 1.0000x reference)
import jax
import jax.numpy as jnp
from jax.experimental import pallas as pl

NUM_HEAD = 4
HEAD_DIM = 64
N_EMBED = 8192
BN = 256


def _vq_body(x_ref, e_ref, q_ref, ind_ref, dsum_ref):
    x = x_ref[...]
    q_cols = []
    ind_cols = []
    for h in range(NUM_HEAD):
        xh = x[:, h * HEAD_DIM:(h + 1) * HEAD_DIM]
        e = e_ref[h]
        esq = jnp.sum(e * e, axis=0, keepdims=True)
        xsq = jnp.sum(xh * xh, axis=1, keepdims=True)
        u = jax.lax.dot(xh, e, preferred_element_type=jnp.float32)
        dist = (xsq - 2.0 * u) + esq
        ind = jnp.argmax(-dist, axis=1).astype(jnp.int32)
        iota_k = jax.lax.broadcasted_iota(jnp.int32, (BN, N_EMBED), 1)
        onehot = (iota_k == ind[:, None]).astype(jnp.float32)
        q = jax.lax.dot_general(
            onehot, e,
            dimension_numbers=(((1,), (1,)), ((), ())),
            precision=jax.lax.Precision.HIGHEST,
            preferred_element_type=jnp.float32,
        )
        q_cols.append(q)
        ind_cols.append(ind)
    q_all = jnp.concatenate(q_cols, axis=1)
    q_ref[...] = q_all
    ind_ref[0] = jnp.stack(ind_cols, axis=1)
    dsum_ref[...] = jnp.sum(jnp.abs(x - q_all)).reshape(1, 1, 1)


@jax.jit
def kernel(input, embed):
    b, s, d = input.shape
    n = b * s
    nb = n // BN
    x2d = input.reshape(n, d)

    q2d, ind3, dsum = pl.pallas_call(
        _vq_body,
        grid=(nb,),
        in_specs=[
            pl.BlockSpec((BN, d), lambda t: (t, 0)),
            pl.BlockSpec((NUM_HEAD, HEAD_DIM, N_EMBED), lambda t: (0, 0, 0)),
        ],
        out_specs=[
            pl.BlockSpec((BN, d), lambda t: (t, 0)),
            pl.BlockSpec((1, BN, NUM_HEAD), lambda t: (t, 0, 0)),
            pl.BlockSpec((1, 1, 1), lambda t: (t, 0, 0)),
        ],
        out_shape=[
            jax.ShapeDtypeStruct((n, d), jnp.float32),
            jax.ShapeDtypeStruct((nb, BN, NUM_HEAD), jnp.int32),
            jax.ShapeDtypeStruct((nb, 1, 1), jnp.float32),
        ],
    )(x2d, embed)

    quantize = q2d.reshape(b, s, d)
    diff = jnp.sum(dsum) / (n * d)
    embed_ind = ind3.reshape(b, s, NUM_HEAD)
    return (quantize, diff, embed_ind)


# speedup vs baseline: 1.5256x; 1.5256x over previous
import functools

import jax
import jax.numpy as jnp
from jax import lax
from jax.experimental import pallas as pl
from jax.experimental.pallas import tpu as pltpu
from jax.experimental.pallas import tpu_sc as plsc

NUM_HEAD = 4
HEAD_DIM = 64
N_EMBED = 8192
BN = 256

NC = 2
NS = 16
NW = NC * NS
ROWS = 16384 * NUM_HEAD
B_PER_W = ROWS // NW
CH = 128
NCHUNK = B_PER_W // CH


def _tc_argmin_body(x_ref, e_ref, ind_ref, find_ref):
    x = x_ref[...]
    ind_cols = []
    for h in range(NUM_HEAD):
        xh = x[:, h * HEAD_DIM:(h + 1) * HEAD_DIM]
        e = e_ref[h]
        esq = jnp.sum(e * e, axis=0, keepdims=True)
        xsq = jnp.sum(xh * xh, axis=1, keepdims=True)
        u = jax.lax.dot(xh, e, preferred_element_type=jnp.float32)
        dist = (xsq - 2.0 * u) + esq
        ind_cols.append(jnp.argmax(-dist, axis=1).astype(jnp.int32))
    ind = jnp.stack(ind_cols, axis=1)
    ind_ref[0] = ind
    off = jax.lax.broadcasted_iota(jnp.int32, (BN, NUM_HEAD), 1) * N_EMBED
    find_ref[0] = ind + off


def _sc_gather_body(table_hbm, idx_hbm, out_hbm, idx_v, rows_v, sem):
    wid = lax.axis_index("s") * NC + lax.axis_index("c")
    base = wid * B_PER_W

    def chunk(c, _):
        start = base + c * CH
        pltpu.sync_copy(idx_hbm.at[pl.ds(start, CH)], idx_v)
        pltpu.async_copy(table_hbm.at[idx_v], rows_v, sem).wait()
        pltpu.sync_copy(rows_v, out_hbm.at[pl.ds(start, CH)])
        return 0

    lax.fori_loop(0, NCHUNK, chunk, 0)


def _make_sc_gather():
    return functools.partial(
        pl.kernel,
        mesh=plsc.VectorSubcoreMesh(core_axis_name="c", subcore_axis_name="s"),
        out_type=jax.ShapeDtypeStruct((ROWS, 128), jnp.float32),
        scratch_types=[
            pltpu.VMEM((CH,), jnp.int32),
            pltpu.VMEM((CH, 128), jnp.float32),
            pltpu.SemaphoreType.DMA,
        ],
    )(_sc_gather_body)


def _tc_l1_body(x_ref, q_ref, dsum_ref):
    dsum_ref[...] = jnp.sum(jnp.abs(x_ref[...] - q_ref[...])).reshape(1, 1, 1)


@jax.jit
def kernel(input, embed):
    b, s, d = input.shape
    n = b * s
    nb = n // BN
    x2d = input.reshape(n, d)

    ind3, find3 = pl.pallas_call(
        _tc_argmin_body,
        grid=(nb,),
        in_specs=[
            pl.BlockSpec((BN, d), lambda t: (t, 0)),
            pl.BlockSpec((NUM_HEAD, HEAD_DIM, N_EMBED), lambda t: (0, 0, 0)),
        ],
        out_specs=[
            pl.BlockSpec((1, BN, NUM_HEAD), lambda t: (t, 0, 0)),
            pl.BlockSpec((1, BN, NUM_HEAD), lambda t: (t, 0, 0)),
        ],
        out_shape=[
            jax.ShapeDtypeStruct((nb, BN, NUM_HEAD), jnp.int32),
            jax.ShapeDtypeStruct((nb, BN, NUM_HEAD), jnp.int32),
        ],
    )(x2d, embed)

    table = embed.transpose(0, 2, 1).reshape(NUM_HEAD * N_EMBED, HEAD_DIM)
    table = jnp.pad(table, ((0, 0), (0, 128 - HEAD_DIM)))
    idx_flat = find3.reshape(ROWS)
    q_flat = _make_sc_gather()(table, idx_flat)

    q2d = q_flat[:, :HEAD_DIM].reshape(n, d)
    dsum = pl.pallas_call(
        _tc_l1_body,
        grid=(nb,),
        in_specs=[
            pl.BlockSpec((BN, d), lambda t: (t, 0)),
            pl.BlockSpec((BN, d), lambda t: (t, 0)),
        ],
        out_specs=pl.BlockSpec((1, 1, 1), lambda t: (t, 0, 0)),
        out_shape=jax.ShapeDtypeStruct((nb, 1, 1), jnp.float32),
    )(x2d, q2d)

    quantize = q2d.reshape(b, s, d)
    diff = jnp.sum(dsum) / (n * d)
    embed_ind = ind3.reshape(b, s, NUM_HEAD)
    return (quantize, diff, embed_ind)


# speedup vs baseline: 1.5875x; 1.0406x over previous
import functools

import jax
import jax.numpy as jnp
from jax import lax
from jax.experimental import pallas as pl
from jax.experimental.pallas import tpu as pltpu
from jax.experimental.pallas import tpu_sc as plsc

NUM_HEAD = 4
HEAD_DIM = 64
N_EMBED = 8192
BN = 256

NC = 2
NS = 16
NW = NC * NS
ROWS = 16384 * NUM_HEAD
B_PER_W = ROWS // NW
CH = 128
NCHUNK = B_PER_W // CH
PADD = 128


def _tc_argmin_body(x_ref, e_ref, ind_ref, find_ref, esq_ref):
    t = pl.program_id(0)

    @pl.when(t == 0)
    def _():
        for h in range(NUM_HEAD):
            e = e_ref[h]
            esq_ref[h] = jnp.sum(e * e, axis=0)

    x = x_ref[...]
    ind_cols = []
    for h in range(NUM_HEAD):
        xh = x[:, h * HEAD_DIM:(h + 1) * HEAD_DIM]
        e = e_ref[h]
        esq = esq_ref[h][None, :]
        xsq = jnp.sum(xh * xh, axis=1, keepdims=True)
        u = jax.lax.dot(xh, e, preferred_element_type=jnp.float32)
        dist = (xsq - 2.0 * u) + esq
        ind_cols.append(jnp.argmax(-dist, axis=1).astype(jnp.int32))
    ind = jnp.stack(ind_cols, axis=1)
    ind_ref[0] = ind
    off = jax.lax.broadcasted_iota(jnp.int32, (BN, NUM_HEAD), 1) * N_EMBED
    find_ref[0] = ind + off


def _sc_gather_body(table_hbm, idx_hbm, out_hbm, idx_all, rows0, rows1,
                    sem0, sem1, wsem0, wsem1):
    wid = lax.axis_index("s") * NC + lax.axis_index("c")
    base = wid * B_PER_W
    pltpu.sync_copy(idx_hbm.at[pl.ds(wid * NCHUNK, NCHUNK)], idx_all)

    rows = [rows0, rows1]
    gsem = [sem0, sem1]
    wsem = [wsem0, wsem1]
    g_h = [None, None]
    wb_h = [None, None]

    def start_gather(c):
        b = c % 2
        if wb_h[b] is not None:
            wb_h[b].wait()
            wb_h[b] = None
        g_h[b] = pltpu.async_copy(table_hbm.at[idx_all.at[c]], rows[b], gsem[b])

    start_gather(0)
    for c in range(NCHUNK):
        if c + 1 < NCHUNK:
            start_gather(c + 1)
        bi = c % 2
        g_h[bi].wait()
        wb_h[bi] = pltpu.async_copy(
            rows[bi], out_hbm.at[pl.ds(base + c * CH, CH)], wsem[bi])
    wb_h[0].wait()
    wb_h[1].wait()


def _make_sc_gather():
    return functools.partial(
        pl.kernel,
        mesh=plsc.VectorSubcoreMesh(core_axis_name="c", subcore_axis_name="s"),
        out_type=jax.ShapeDtypeStruct((ROWS, PADD), jnp.float32),
        scratch_types=[
            pltpu.VMEM((NCHUNK, CH), jnp.int32),
            pltpu.VMEM((CH, PADD), jnp.float32),
            pltpu.VMEM((CH, PADD), jnp.float32),
            pltpu.SemaphoreType.DMA,
            pltpu.SemaphoreType.DMA,
            pltpu.SemaphoreType.DMA,
            pltpu.SemaphoreType.DMA,
        ],
    )(_sc_gather_body)


def _tc_l1_body(x_ref, q_ref, dsum_ref):
    dsum_ref[...] = jnp.sum(jnp.abs(x_ref[...] - q_ref[...])).reshape(1, 1, 1)


@jax.jit
def kernel(input, embed):
    b, s, d = input.shape
    n = b * s
    nb = n // BN
    x2d = input.reshape(n, d)

    ind3, find3 = pl.pallas_call(
        _tc_argmin_body,
        grid=(nb,),
        in_specs=[
            pl.BlockSpec((BN, d), lambda t: (t, 0)),
            pl.BlockSpec((NUM_HEAD, HEAD_DIM, N_EMBED), lambda t: (0, 0, 0)),
        ],
        out_specs=[
            pl.BlockSpec((1, BN, NUM_HEAD), lambda t: (t, 0, 0)),
            pl.BlockSpec((1, BN, NUM_HEAD), lambda t: (t, 0, 0)),
        ],
        out_shape=[
            jax.ShapeDtypeStruct((nb, BN, NUM_HEAD), jnp.int32),
            jax.ShapeDtypeStruct((nb, BN, NUM_HEAD), jnp.int32),
        ],
        scratch_shapes=[pltpu.VMEM((NUM_HEAD, N_EMBED), jnp.float32)],
    )(x2d, embed)

    table = embed.transpose(0, 2, 1).reshape(NUM_HEAD * N_EMBED, HEAD_DIM)
    table = jnp.pad(table, ((0, 0), (0, PADD - HEAD_DIM)))
    idx_flat = find3.reshape(NW * NCHUNK, CH)
    q_flat = _make_sc_gather()(table, idx_flat)

    q2d = q_flat[:, :HEAD_DIM].reshape(n, d)
    dsum = pl.pallas_call(
        _tc_l1_body,
        grid=(nb,),
        in_specs=[
            pl.BlockSpec((BN, d), lambda t: (t, 0)),
            pl.BlockSpec((BN, d), lambda t: (t, 0)),
        ],
        out_specs=pl.BlockSpec((1, 1, 1), lambda t: (t, 0, 0)),
        out_shape=jax.ShapeDtypeStruct((nb, 1, 1), jnp.float32),
    )(x2d, q2d)

    quantize = q2d.reshape(b, s, d)
    diff = jnp.sum(dsum) / (n * d)
    embed_ind = ind3.reshape(b, s, NUM_HEAD)
    return (quantize, diff, embed_ind)


# speedup vs baseline: 1.5875x; 1.0000x over previous
import functools

import jax
import jax.numpy as jnp
from jax import lax
from jax.experimental import pallas as pl
from jax.experimental.pallas import tpu as pltpu
from jax.experimental.pallas import tpu_sc as plsc

NUM_HEAD = 4
HEAD_DIM = 64
N_EMBED = 8192
BN = 256

NC = 2
NS = 16
NW = NC * NS
ROWS = 16384 * NUM_HEAD
B_PER_W = ROWS // NW
CH = 128
NCHUNK = B_PER_W // CH
PADD = 128


def _tc_argmin_body(x_ref, e_ref, ind_ref, find_ref, esq_ref):
    t = pl.program_id(0)

    @pl.when(t == 0)
    def _():
        for h in range(NUM_HEAD):
            e = e_ref[h]
            esq_ref[h] = jnp.sum(e * e, axis=0)

    x = x_ref[...]
    ind_cols = []
    for h in range(NUM_HEAD):
        xh = x[:, h * HEAD_DIM:(h + 1) * HEAD_DIM]
        e = e_ref[h]
        esq = esq_ref[h][None, :]
        xsq = jnp.sum(xh * xh, axis=1, keepdims=True)
        u = jax.lax.dot(xh, e, preferred_element_type=jnp.float32)
        dist = (xsq - 2.0 * u) + esq
        ind_cols.append(jnp.argmax(-dist, axis=1).astype(jnp.int32))
    ind = jnp.stack(ind_cols, axis=1)
    ind_ref[0] = ind
    off = jax.lax.broadcasted_iota(jnp.int32, (BN, NUM_HEAD), 1) * N_EMBED
    find_ref[0] = ind + off


def _sc_gather_body(table_hbm, idx_hbm, out_hbm, idx_all, rows0, rows1, rows2,
                    rows3, sem0, sem1, sem2, sem3, wsem0, wsem1, wsem2, wsem3):
    wid = lax.axis_index("s") * NC + lax.axis_index("c")
    base = wid * B_PER_W
    pltpu.sync_copy(idx_hbm.at[pl.ds(wid * NCHUNK, NCHUNK)], idx_all)

    rows = [rows0, rows1, rows2, rows3]
    gsem = [sem0, sem1, sem2, sem3]
    wsem = [wsem0, wsem1, wsem2, wsem3]
    g_h = [None] * 4
    wb_h = [None] * 4

    def start_gather(c):
        b = c % 4
        if wb_h[b] is not None:
            wb_h[b].wait()
            wb_h[b] = None
        g_h[b] = pltpu.async_copy(table_hbm.at[idx_all.at[c]], rows[b], gsem[b])

    start_gather(0)
    start_gather(1)
    start_gather(2)
    for c in range(NCHUNK):
        if c + 3 < NCHUNK:
            start_gather(c + 3)
        bi = c % 4
        g_h[bi].wait()
        wb_h[bi] = pltpu.async_copy(
            rows[bi], out_hbm.at[pl.ds(base + c * CH, CH)], wsem[bi])
    for bi in range(4):
        if wb_h[bi] is not None:
            wb_h[bi].wait()


def _make_sc_gather():
    return functools.partial(
        pl.kernel,
        mesh=plsc.VectorSubcoreMesh(core_axis_name="c", subcore_axis_name="s"),
        out_type=jax.ShapeDtypeStruct((ROWS, PADD), jnp.float32),
        scratch_types=[
            pltpu.VMEM((NCHUNK, CH), jnp.int32),
            pltpu.VMEM((CH, PADD), jnp.float32),
            pltpu.VMEM((CH, PADD), jnp.float32),
            pltpu.VMEM((CH, PADD), jnp.float32),
            pltpu.VMEM((CH, PADD), jnp.float32),
            pltpu.SemaphoreType.DMA,
            pltpu.SemaphoreType.DMA,
            pltpu.SemaphoreType.DMA,
            pltpu.SemaphoreType.DMA,
            pltpu.SemaphoreType.DMA,
            pltpu.SemaphoreType.DMA,
            pltpu.SemaphoreType.DMA,
            pltpu.SemaphoreType.DMA,
        ],
    )(_sc_gather_body)


def _tc_l1_body(x_ref, q_ref, dsum_ref):
    dsum_ref[...] = jnp.sum(jnp.abs(x_ref[...] - q_ref[...])).reshape(1, 1, 1)


@jax.jit
def kernel(input, embed):
    b, s, d = input.shape
    n = b * s
    nb = n // BN
    x2d = input.reshape(n, d)

    ind3, find3 = pl.pallas_call(
        _tc_argmin_body,
        grid=(nb,),
        in_specs=[
            pl.BlockSpec((BN, d), lambda t: (t, 0)),
            pl.BlockSpec((NUM_HEAD, HEAD_DIM, N_EMBED), lambda t: (0, 0, 0)),
        ],
        out_specs=[
            pl.BlockSpec((1, BN, NUM_HEAD), lambda t: (t, 0, 0)),
            pl.BlockSpec((1, BN, NUM_HEAD), lambda t: (t, 0, 0)),
        ],
        out_shape=[
            jax.ShapeDtypeStruct((nb, BN, NUM_HEAD), jnp.int32),
            jax.ShapeDtypeStruct((nb, BN, NUM_HEAD), jnp.int32),
        ],
        scratch_shapes=[pltpu.VMEM((NUM_HEAD, N_EMBED), jnp.float32)],
    )(x2d, embed)

    table = embed.transpose(0, 2, 1).reshape(NUM_HEAD * N_EMBED, HEAD_DIM)
    table = jnp.pad(table, ((0, 0), (0, PADD - HEAD_DIM)))
    idx_flat = find3.reshape(NW * NCHUNK, CH)
    q_flat = _make_sc_gather()(table, idx_flat)

    q2d = q_flat[:, :HEAD_DIM].reshape(n, d)
    dsum = pl.pallas_call(
        _tc_l1_body,
        grid=(nb,),
        in_specs=[
            pl.BlockSpec((BN, d), lambda t: (t, 0)),
            pl.BlockSpec((BN, d), lambda t: (t, 0)),
        ],
        out_specs=pl.BlockSpec((1, 1, 1), lambda t: (t, 0, 0)),
        out_shape=jax.ShapeDtypeStruct((nb, 1, 1), jnp.float32),
    )(x2d, q2d)

    quantize = q2d.reshape(b, s, d)
    diff = jnp.sum(dsum) / (n * d)
    embed_ind = ind3.reshape(b, s, NUM_HEAD)
    return (quantize, diff, embed_ind)


# speedup vs baseline: 1.5910x; 1.0022x over previous
import functools

import jax
import jax.numpy as jnp
from jax import lax
from jax.experimental import pallas as pl
from jax.experimental.pallas import tpu as pltpu
from jax.experimental.pallas import tpu_sc as plsc

NUM_HEAD = 4
HEAD_DIM = 64
N_EMBED = 8192
BN = 256

NC = 2
NS = 16
NW = NC * NS
ROWS = 16384 * NUM_HEAD
B_PER_W = ROWS // NW
CH = 128
NCHUNK = B_PER_W // CH
PADD = 128


def _tc_argmin_body(x_ref, e_ref, ind_ref, find_ref, esq_ref):
    t = pl.program_id(0)

    @pl.when(t == 0)
    def _():
        for h in range(NUM_HEAD):
            e = e_ref[h]
            esq_ref[h] = jnp.sum(e * e, axis=0)

    x = x_ref[...]
    ind_cols = []
    for h in range(NUM_HEAD):
        xh = x[:, h * HEAD_DIM:(h + 1) * HEAD_DIM]
        e = e_ref[h]
        esq = esq_ref[h][None, :]
        xsq = jnp.sum(xh * xh, axis=1, keepdims=True)
        u = jax.lax.dot(xh, e, preferred_element_type=jnp.float32)
        dist = (xsq - 2.0 * u) + esq
        ind_cols.append(jnp.argmax(-dist, axis=1).astype(jnp.int32))
    ind = jnp.stack(ind_cols, axis=1)
    ind_ref[0] = ind
    off = jax.lax.broadcasted_iota(jnp.int32, (BN, NUM_HEAD), 1) * N_EMBED
    find_ref[0] = ind + off


def _sc_gather_body(table_hbm, idx_hbm, out_hbm, idx_all, rows0, rows1, rows2,
                    rows3, sem0, sem1, sem2, sem3, wsem0, wsem1, wsem2, wsem3):
    wid = lax.axis_index("s") * NC + lax.axis_index("c")
    base = wid * B_PER_W
    pltpu.sync_copy(idx_hbm.at[pl.ds(wid * NCHUNK, NCHUNK)], idx_all)

    rows = [rows0, rows1, rows2, rows3]
    gsem = [sem0, sem1, sem2, sem3]
    wsem = [wsem0, wsem1, wsem2, wsem3]
    g_h = [None] * 4
    wb_h = [None] * 4

    def start_gather(c):
        b = c % 4
        if wb_h[b] is not None:
            wb_h[b].wait()
            wb_h[b] = None
        g_h[b] = pltpu.async_copy(table_hbm.at[idx_all.at[c]], rows[b], gsem[b])

    start_gather(0)
    start_gather(1)
    start_gather(2)
    for c in range(NCHUNK):
        if c + 3 < NCHUNK:
            start_gather(c + 3)
        bi = c % 4
        g_h[bi].wait()
        wb_h[bi] = pltpu.async_copy(
            rows[bi], out_hbm.at[pl.ds(base + c * CH, CH)], wsem[bi])
    for bi in range(4):
        if wb_h[bi] is not None:
            wb_h[bi].wait()


def _make_sc_gather():
    return functools.partial(
        pl.kernel,
        mesh=plsc.VectorSubcoreMesh(core_axis_name="c", subcore_axis_name="s"),
        compiler_params=pltpu.CompilerParams(use_tc_tiling_on_sc=False),
        out_type=jax.ShapeDtypeStruct((ROWS, PADD), jnp.float32),
        scratch_types=[
            pltpu.VMEM((NCHUNK, CH), jnp.int32),
            pltpu.VMEM((CH, PADD), jnp.float32),
            pltpu.VMEM((CH, PADD), jnp.float32),
            pltpu.VMEM((CH, PADD), jnp.float32),
            pltpu.VMEM((CH, PADD), jnp.float32),
            pltpu.SemaphoreType.DMA,
            pltpu.SemaphoreType.DMA,
            pltpu.SemaphoreType.DMA,
            pltpu.SemaphoreType.DMA,
            pltpu.SemaphoreType.DMA,
            pltpu.SemaphoreType.DMA,
            pltpu.SemaphoreType.DMA,
            pltpu.SemaphoreType.DMA,
        ],
    )(_sc_gather_body)


def _tc_l1_body(x_ref, q_ref, dsum_ref):
    dsum_ref[...] = jnp.sum(jnp.abs(x_ref[...] - q_ref[...])).reshape(1, 1, 1)


@jax.jit
def kernel(input, embed):
    b, s, d = input.shape
    n = b * s
    nb = n // BN
    x2d = input.reshape(n, d)

    ind3, find3 = pl.pallas_call(
        _tc_argmin_body,
        grid=(nb,),
        in_specs=[
            pl.BlockSpec((BN, d), lambda t: (t, 0)),
            pl.BlockSpec((NUM_HEAD, HEAD_DIM, N_EMBED), lambda t: (0, 0, 0)),
        ],
        out_specs=[
            pl.BlockSpec((1, BN, NUM_HEAD), lambda t: (t, 0, 0)),
            pl.BlockSpec((1, BN, NUM_HEAD), lambda t: (t, 0, 0)),
        ],
        out_shape=[
            jax.ShapeDtypeStruct((nb, BN, NUM_HEAD), jnp.int32),
            jax.ShapeDtypeStruct((nb, BN, NUM_HEAD), jnp.int32),
        ],
        scratch_shapes=[pltpu.VMEM((NUM_HEAD, N_EMBED), jnp.float32)],
    )(x2d, embed)

    table = embed.transpose(0, 2, 1).reshape(NUM_HEAD * N_EMBED, HEAD_DIM)
    table = jnp.pad(table, ((0, 0), (0, PADD - HEAD_DIM)))
    idx_flat = find3.reshape(NW * NCHUNK, CH)
    q_flat = _make_sc_gather()(table, idx_flat)

    q2d = q_flat[:, :HEAD_DIM].reshape(n, d)
    dsum = pl.pallas_call(
        _tc_l1_body,
        grid=(nb,),
        in_specs=[
            pl.BlockSpec((BN, d), lambda t: (t, 0)),
            pl.BlockSpec((BN, d), lambda t: (t, 0)),
        ],
        out_specs=pl.BlockSpec((1, 1, 1), lambda t: (t, 0, 0)),
        out_shape=jax.ShapeDtypeStruct((nb, 1, 1), jnp.float32),
    )(x2d, q2d)

    quantize = q2d.reshape(b, s, d)
    diff = jnp.sum(dsum) / (n * d)
    embed_ind = ind3.reshape(b, s, NUM_HEAD)
    return (quantize, diff, embed_ind)


# speedup vs baseline: 2.2177x; 1.3939x over previous
import functools

import jax
import jax.numpy as jnp
from jax import lax
from jax.experimental import pallas as pl
from jax.experimental.pallas import tpu as pltpu
from jax.experimental.pallas import tpu_sc as plsc

NUM_HEAD = 4
HEAD_DIM = 64
N_EMBED = 8192
BN = 256

NC = 2
NS = 16
NW = NC * NS
ROWS = 16384 * NUM_HEAD
NSLICE = 4
SROWS = ROWS // NSLICE
B_PER_W = SROWS // NW
CH = 128
NCHUNK = B_PER_W // CH
PADD = 128


def _tc_argmin_body(x_ref, e_ref, ind_ref, find_ref, esq_ref):
    t = pl.program_id(0)

    @pl.when(t == 0)
    def _():
        for h in range(NUM_HEAD):
            e = e_ref[h]
            esq_ref[h] = jnp.sum(e * e, axis=0)

    x = x_ref[...]
    ind_cols = []
    for h in range(NUM_HEAD):
        xh = x[:, h * HEAD_DIM:(h + 1) * HEAD_DIM]
        e = e_ref[h]
        esq = esq_ref[h][None, :]
        xsq = jnp.sum(xh * xh, axis=1, keepdims=True)
        u = jax.lax.dot(xh, e, preferred_element_type=jnp.float32)
        dist = (xsq - 2.0 * u) + esq
        ind_cols.append(jnp.argmax(-dist, axis=1).astype(jnp.int32))
    ind = jnp.stack(ind_cols, axis=1)
    ind_ref[0] = ind
    off = jax.lax.broadcasted_iota(jnp.int32, (BN, NUM_HEAD), 1) * N_EMBED
    find_ref[0] = ind + off


def _sc_gather_body(table_hbm, idx_hbm, out_hbm, idx_all, rows0, rows1, rows2,
                    rows3, sem0, sem1, sem2, sem3, wsem0, wsem1, wsem2, wsem3):
    wid = lax.axis_index("s") * NC + lax.axis_index("c")
    base = wid * B_PER_W
    pltpu.sync_copy(idx_hbm.at[pl.ds(wid * NCHUNK, NCHUNK)], idx_all)

    rows = [rows0, rows1, rows2, rows3]
    gsem = [sem0, sem1, sem2, sem3]
    wsem = [wsem0, wsem1, wsem2, wsem3]
    g_h = [None] * 4
    wb_h = [None] * 4

    def start_gather(c):
        b = c % 4
        if wb_h[b] is not None:
            wb_h[b].wait()
            wb_h[b] = None
        g_h[b] = pltpu.async_copy(table_hbm.at[idx_all.at[c]], rows[b], gsem[b])

    start_gather(0)
    start_gather(1)
    start_gather(2)
    for c in range(NCHUNK):
        if c + 3 < NCHUNK:
            start_gather(c + 3)
        bi = c % 4
        g_h[bi].wait()
        wb_h[bi] = pltpu.async_copy(
            rows[bi], out_hbm.at[pl.ds(base + c * CH, CH)], wsem[bi])
    for bi in range(4):
        if wb_h[bi] is not None:
            wb_h[bi].wait()


def _make_sc_gather():
    return functools.partial(
        pl.kernel,
        mesh=plsc.VectorSubcoreMesh(core_axis_name="c", subcore_axis_name="s"),
        out_type=jax.ShapeDtypeStruct((SROWS, PADD), jnp.float32),
        scratch_types=[
            pltpu.VMEM((NCHUNK, CH), jnp.int32),
            pltpu.VMEM((CH, PADD), jnp.float32),
            pltpu.VMEM((CH, PADD), jnp.float32),
            pltpu.VMEM((CH, PADD), jnp.float32),
            pltpu.VMEM((CH, PADD), jnp.float32),
            pltpu.SemaphoreType.DMA,
            pltpu.SemaphoreType.DMA,
            pltpu.SemaphoreType.DMA,
            pltpu.SemaphoreType.DMA,
            pltpu.SemaphoreType.DMA,
            pltpu.SemaphoreType.DMA,
            pltpu.SemaphoreType.DMA,
            pltpu.SemaphoreType.DMA,
        ],
    )(_sc_gather_body)


def _tc_l1_body(x_ref, q_ref, dsum_ref):
    dsum_ref[...] = jnp.sum(jnp.abs(x_ref[...] - q_ref[...])).reshape(1, 1, 1)


@jax.jit
def kernel(input, embed):
    b, s, d = input.shape
    n = b * s
    ns = n // NSLICE
    nbs = ns // BN
    x2d = input.reshape(n, d)

    table = embed.transpose(0, 2, 1).reshape(NUM_HEAD * N_EMBED, HEAD_DIM)
    table = jnp.pad(table, ((0, 0), (0, PADD - HEAD_DIM)))

    sc_gather = _make_sc_gather()
    tc_argmin = pl.pallas_call(
        _tc_argmin_body,
        grid=(nbs,),
        in_specs=[
            pl.BlockSpec((BN, d), lambda t: (t, 0)),
            pl.BlockSpec((NUM_HEAD, HEAD_DIM, N_EMBED), lambda t: (0, 0, 0)),
        ],
        out_specs=[
            pl.BlockSpec((1, BN, NUM_HEAD), lambda t: (t, 0, 0)),
            pl.BlockSpec((1, BN, NUM_HEAD), lambda t: (t, 0, 0)),
        ],
        out_shape=[
            jax.ShapeDtypeStruct((nbs, BN, NUM_HEAD), jnp.int32),
            jax.ShapeDtypeStruct((nbs, BN, NUM_HEAD), jnp.int32),
        ],
        scratch_shapes=[pltpu.VMEM((NUM_HEAD, N_EMBED), jnp.float32)],
    )

    ind_slices = []
    q_slices = []
    for si in range(NSLICE):
        xs = jax.lax.slice_in_dim(x2d, si * ns, (si + 1) * ns, axis=0)
        ind3, find3 = tc_argmin(xs, embed)
        q_flat = sc_gather(table, find3.reshape(NW * NCHUNK, CH))
        ind_slices.append(ind3)
        q_slices.append(q_flat[:, :HEAD_DIM])

    ind3 = jnp.concatenate(ind_slices, axis=0)
    q2d = jnp.concatenate(q_slices, axis=0).reshape(n, d)
    nb = n // BN
    dsum = pl.pallas_call(
        _tc_l1_body,
        grid=(nb,),
        in_specs=[
            pl.BlockSpec((BN, d), lambda t: (t, 0)),
            pl.BlockSpec((BN, d), lambda t: (t, 0)),
        ],
        out_specs=pl.BlockSpec((1, 1, 1), lambda t: (t, 0, 0)),
        out_shape=jax.ShapeDtypeStruct((nb, 1, 1), jnp.float32),
    )(x2d, q2d)

    quantize = q2d.reshape(b, s, d)
    diff = jnp.sum(dsum) / (n * d)
    embed_ind = ind3.reshape(b, s, NUM_HEAD)
    return (quantize, diff, embed_ind)


# speedup vs baseline: 2.4200x; 1.0912x over previous
import functools

import jax
import jax.numpy as jnp
from jax import lax
from jax.experimental import pallas as pl
from jax.experimental.pallas import tpu as pltpu
from jax.experimental.pallas import tpu_sc as plsc

NUM_HEAD = 4
HEAD_DIM = 64
N_EMBED = 8192
BN = 256

NC = 2
NS = 16
NW = NC * NS
ROWS = 16384 * NUM_HEAD
NSLICE = 8
SROWS = ROWS // NSLICE
B_PER_W = SROWS // NW
CH = 128
NCHUNK = B_PER_W // CH
PADD = 128


def _tc_argmin_body(x_ref, e_ref, ind_ref, find_ref, esq_ref):
    t = pl.program_id(0)

    @pl.when(t == 0)
    def _():
        for h in range(NUM_HEAD):
            e = e_ref[h]
            esq_ref[h] = jnp.sum(e * e, axis=0)

    x = x_ref[...]
    ind_cols = []
    for h in range(NUM_HEAD):
        xh = x[:, h * HEAD_DIM:(h + 1) * HEAD_DIM]
        e = e_ref[h]
        esq = esq_ref[h][None, :]
        xsq = jnp.sum(xh * xh, axis=1, keepdims=True)
        u = jax.lax.dot(xh, e, preferred_element_type=jnp.float32)
        dist = (xsq - 2.0 * u) + esq
        ind_cols.append(jnp.argmax(-dist, axis=1).astype(jnp.int32))
    ind = jnp.stack(ind_cols, axis=1)
    ind_ref[0] = ind
    off = jax.lax.broadcasted_iota(jnp.int32, (BN, NUM_HEAD), 1) * N_EMBED
    find_ref[0] = ind + off


def _sc_gather_body(table_hbm, idx_hbm, out_hbm, idx_all, rows0, rows1, rows2,
                    rows3, sem0, sem1, sem2, sem3, wsem0, wsem1, wsem2, wsem3):
    wid = lax.axis_index("s") * NC + lax.axis_index("c")
    base = wid * B_PER_W
    pltpu.sync_copy(idx_hbm.at[pl.ds(wid * NCHUNK, NCHUNK)], idx_all)

    rows = [rows0, rows1, rows2, rows3]
    gsem = [sem0, sem1, sem2, sem3]
    wsem = [wsem0, wsem1, wsem2, wsem3]
    g_h = [None] * 4
    wb_h = [None] * 4

    def start_gather(c):
        b = c % 4
        if wb_h[b] is not None:
            wb_h[b].wait()
            wb_h[b] = None
        g_h[b] = pltpu.async_copy(table_hbm.at[idx_all.at[c]], rows[b], gsem[b])

    for c in range(min(3, NCHUNK)):
        start_gather(c)
    for c in range(NCHUNK):
        if c + 3 < NCHUNK:
            start_gather(c + 3)
        bi = c % 4
        g_h[bi].wait()
        wb_h[bi] = pltpu.async_copy(
            rows[bi], out_hbm.at[pl.ds(base + c * CH, CH)], wsem[bi])
    for bi in range(4):
        if wb_h[bi] is not None:
            wb_h[bi].wait()


def _make_sc_gather():
    return functools.partial(
        pl.kernel,
        mesh=plsc.VectorSubcoreMesh(core_axis_name="c", subcore_axis_name="s"),
        out_type=jax.ShapeDtypeStruct((SROWS, PADD), jnp.float32),
        scratch_types=[
            pltpu.VMEM((NCHUNK, CH), jnp.int32),
            pltpu.VMEM((CH, PADD), jnp.float32),
            pltpu.VMEM((CH, PADD), jnp.float32),
            pltpu.VMEM((CH, PADD), jnp.float32),
            pltpu.VMEM((CH, PADD), jnp.float32),
            pltpu.SemaphoreType.DMA,
            pltpu.SemaphoreType.DMA,
            pltpu.SemaphoreType.DMA,
            pltpu.SemaphoreType.DMA,
            pltpu.SemaphoreType.DMA,
            pltpu.SemaphoreType.DMA,
            pltpu.SemaphoreType.DMA,
            pltpu.SemaphoreType.DMA,
        ],
    )(_sc_gather_body)


def _tc_l1_body(x_ref, q_ref, dsum_ref):
    dsum_ref[...] = jnp.sum(jnp.abs(x_ref[...] - q_ref[...])).reshape(1, 1, 1)


@jax.jit
def kernel(input, embed):
    b, s, d = input.shape
    n = b * s
    ns = n // NSLICE
    nbs = ns // BN
    x2d = input.reshape(n, d)

    table = embed.transpose(0, 2, 1).reshape(NUM_HEAD * N_EMBED, HEAD_DIM)
    table = jnp.pad(table, ((0, 0), (0, PADD - HEAD_DIM)))

    sc_gather = _make_sc_gather()
    tc_argmin = pl.pallas_call(
        _tc_argmin_body,
        grid=(nbs,),
        in_specs=[
            pl.BlockSpec((BN, d), lambda t: (t, 0)),
            pl.BlockSpec((NUM_HEAD, HEAD_DIM, N_EMBED), lambda t: (0, 0, 0)),
        ],
        out_specs=[
            pl.BlockSpec((1, BN, NUM_HEAD), lambda t: (t, 0, 0)),
            pl.BlockSpec((1, BN, NUM_HEAD), lambda t: (t, 0, 0)),
        ],
        out_shape=[
            jax.ShapeDtypeStruct((nbs, BN, NUM_HEAD), jnp.int32),
            jax.ShapeDtypeStruct((nbs, BN, NUM_HEAD), jnp.int32),
        ],
        scratch_shapes=[pltpu.VMEM((NUM_HEAD, N_EMBED), jnp.float32)],
    )

    nbs_l1 = ns // BN
    tc_l1 = pl.pallas_call(
        _tc_l1_body,
        grid=(nbs_l1,),
        in_specs=[
            pl.BlockSpec((BN, d), lambda t: (t, 0)),
            pl.BlockSpec((BN, d), lambda t: (t, 0)),
        ],
        out_specs=pl.BlockSpec((1, 1, 1), lambda t: (t, 0, 0)),
        out_shape=jax.ShapeDtypeStruct((nbs_l1, 1, 1), jnp.float32),
    )

    ind_slices = []
    q_slices = []
    dsum_slices = []
    for si in range(NSLICE):
        xs = jax.lax.slice_in_dim(x2d, si * ns, (si + 1) * ns, axis=0)
        ind3, find3 = tc_argmin(xs, embed)
        q_flat = sc_gather(table, find3.reshape(NW * NCHUNK, CH))
        qs = q_flat[:, :HEAD_DIM].reshape(ns, d)
        dsum_slices.append(tc_l1(xs, qs))
        ind_slices.append(ind3)
        q_slices.append(qs)

    ind3 = jnp.concatenate(ind_slices, axis=0)
    q2d = jnp.concatenate(q_slices, axis=0)
    dsum = jnp.stack(dsum_slices)

    quantize = q2d.reshape(b, s, d)
    diff = jnp.sum(dsum) / (n * d)
    embed_ind = ind3.reshape(b, s, NUM_HEAD)
    return (quantize, diff, embed_ind)
